# Initial kernel scaffold; baseline (speedup 1.0000x reference)
import jax
import jax.numpy as jnp
from jax.experimental import pallas as pl


def kernel(cls0, cls1, cls2, cls3, cls4, reg0, reg1, reg2, reg3, reg4, images_shape):
    raise NotImplementedError("write your pallas kernel here")


# speedup vs baseline: 35.9036x; 35.9036x over previous
import numpy as np
import jax
import jax.numpy as jnp
from jax.experimental import pallas as pl
from jax.experimental.pallas import tpu as pltpu

_SCALES = np.array([8.0])
_RATIOS = np.array([0.5, 1.0, 2.0])
_STRIDES = [4, 8, 16, 32, 64]
_IMG_H, _IMG_W = 512, 512
_PRE_NMS = 2000
_MAX_PER_IMG = 1000
_NMS_THR = 0.7
_MAX_RATIO = abs(float(np.log(16.0 / 1000.0)))

_BLK = 128
_NB = 56
_N_PAD = _NB * _BLK
_N_REAL = 2000 + 2000 + 2000 + 768 + 192


def _grids_np():
    grids = []
    for stride in _STRIDES:
        H = _IMG_H // stride
        W = _IMG_W // stride
        hr = np.sqrt(_RATIOS)
        wr = 1.0 / hr
        ws = (stride * wr[:, None] * _SCALES[None, :]).reshape(-1)
        hs = (stride * hr[:, None] * _SCALES[None, :]).reshape(-1)
        base = np.stack([-0.5 * ws, -0.5 * hs, 0.5 * ws, 0.5 * hs], axis=1)
        sx = np.arange(W) * stride
        sy = np.arange(H) * stride
        yy, xx = np.meshgrid(sy, sx, indexing='ij')
        shifts = np.stack([xx.ravel(), yy.ravel(), xx.ravel(), yy.ravel()], axis=1)
        anchors = (shifts[:, None, :].astype(np.float32)
                   + base[None, :, :].astype(np.float32)).reshape(-1, 4)
        grids.append(anchors)
    return grids


_GRIDS = _grids_np()


def _decode_body(ax1_ref, ay1_ref, ax2_ref, ay2_ref,
                 dx_ref, dy_ref, dw_ref, dh_ref, s_ref, lvl_ref,
                 x1_ref, y1_ref, x2_ref, y2_ref,
                 nx1_ref, ny1_ref, nx2_ref, ny2_ref, sm_ref):
    ax1 = ax1_ref[...]
    ay1 = ay1_ref[...]
    ax2 = ax2_ref[...]
    ay2 = ay2_ref[...]
    px = (ax1 + ax2) * 0.5
    py = (ay1 + ay2) * 0.5
    pw = ax2 - ax1
    ph = ay2 - ay1
    dx = dx_ref[...]
    dy = dy_ref[...]
    dw = jnp.clip(dw_ref[...], -_MAX_RATIO, _MAX_RATIO)
    dh = jnp.clip(dh_ref[...], -_MAX_RATIO, _MAX_RATIO)
    gx = px + pw * dx
    gy = py + ph * dy
    gw = pw * jnp.exp(dw)
    gh = ph * jnp.exp(dh)
    x1 = jnp.clip(gx - 0.5 * gw, 0.0, float(_IMG_W))
    y1 = jnp.clip(gy - 0.5 * gh, 0.0, float(_IMG_H))
    x2 = jnp.clip(gx + 0.5 * gw, 0.0, float(_IMG_W))
    y2 = jnp.clip(gy + 0.5 * gh, 0.0, float(_IMG_H))
    valid = (x2 > x1) & (y2 > y1)
    sm = jnp.where(valid, s_ref[...], -1.0)
    max_c = jnp.max(jnp.maximum(jnp.maximum(x1, y1), jnp.maximum(x2, y2)))
    off = lvl_ref[...] * (max_c + 1.0)
    x1_ref[...] = x1
    y1_ref[...] = y1
    x2_ref[...] = x2
    y2_ref[...] = y2
    nx1_ref[...] = x1 + off
    ny1_ref[...] = y1 + off
    nx2_ref[...] = x2 + off
    ny2_ref[...] = y2 + off
    sm_ref[...] = sm


def _nms_body(x1_ref, y1_ref, x2_ref, y2_ref, keep_ref, mt_ref):
    keep_ref[...] = jnp.ones((_NB, _BLK), jnp.float32)
    col = jax.lax.broadcasted_iota(jnp.int32, (1, _BLK), 1)
    rowi = jax.lax.broadcasted_iota(jnp.int32, (_BLK, _BLK), 0)
    coli = jax.lax.broadcasted_iota(jnp.int32, (_BLK, _BLK), 1)
    tri = jnp.where(rowi < coli, 1.0, 0.0)

    def load_row(ref, b):
        return ref[pl.ds(b, 1), :]

    def iou(a1, b1, a2, b2, aa, c1, d1, c2, d2, ca):
        xx1 = jnp.maximum(a1, c1)
        yy1 = jnp.maximum(b1, d1)
        xx2 = jnp.minimum(a2, c2)
        yy2 = jnp.minimum(b2, d2)
        inter = jnp.maximum(xx2 - xx1, 0.0) * jnp.maximum(yy2 - yy1, 0.0)
        return inter / (aa + ca - inter)

    def outer(b, carry):
        bx1 = load_row(x1_ref, b)
        by1 = load_row(y1_ref, b)
        bx2 = load_row(x2_ref, b)
        by2 = load_row(y2_ref, b)
        ba = (bx2 - bx1) * (by2 - by1)
        cx1 = bx1.reshape(_BLK, 1)
        cy1 = by1.reshape(_BLK, 1)
        cx2 = bx2.reshape(_BLK, 1)
        cy2 = by2.reshape(_BLK, 1)
        cba = ba.reshape(_BLK, 1)
        m = iou(cx1, cy1, cx2, cy2, cba, bx1, by1, bx2, by2, ba)
        mt_ref[...] = jnp.where(m > _NMS_THR, 1.0, 0.0) * tri

        def inner(i, k):
            ki = jnp.sum(jnp.where(col == i, k, 0.0))
            mrow = mt_ref[pl.ds(i, 1), :]
            return k * (1.0 - mrow * ki)

        kb = jax.lax.fori_loop(0, _BLK, inner,
                               load_row(keep_ref, b))
        keep_ref[pl.ds(b, 1), :] = kb
        kcol = kb.reshape(_BLK, 1)

        def cross(b2, c2_):
            ox1 = load_row(x1_ref, b2)
            oy1 = load_row(y1_ref, b2)
            ox2 = load_row(x2_ref, b2)
            oy2 = load_row(y2_ref, b2)
            oa = (ox2 - ox1) * (oy2 - oy1)
            m2 = iou(cx1, cy1, cx2, cy2, cba, ox1, oy1, ox2, oy2, oa)
            sup = jnp.max(jnp.where(m2 > _NMS_THR, kcol, 0.0),
                          axis=0, keepdims=True)
            kr = load_row(keep_ref, b2)
            keep_ref[pl.ds(b2, 1), :] = kr * (1.0 - sup)
            return c2_

        jax.lax.fori_loop(b + 1, _NB, cross, 0)
        return carry

    jax.lax.fori_loop(0, _NB, outer, 0)


def _decode_call(ax1, ay1, ax2, ay2, dx, dy, dw, dh, s, lvl):
    shp = jax.ShapeDtypeStruct((_NB, _BLK), jnp.float32)
    return pl.pallas_call(
        _decode_body,
        out_shape=(shp,) * 9,
    )(ax1, ay1, ax2, ay2, dx, dy, dw, dh, s, lvl)


def _nms_call(nx1, ny1, nx2, ny2):
    return pl.pallas_call(
        _nms_body,
        out_shape=jax.ShapeDtypeStruct((_NB, _BLK), jnp.float32),
        scratch_shapes=[pltpu.VMEM((_BLK, _BLK), jnp.float32)],
    )(nx1, ny1, nx2, ny2)


def kernel(cls0, cls1, cls2, cls3, cls4, reg0, reg1, reg2, reg3, reg4,
           images_shape):
    cls_list = [cls0, cls1, cls2, cls3, cls4]
    reg_list = [reg0, reg1, reg2, reg3, reg4]
    B = cls0.shape[0]

    s_parts, d_parts, a_parts, lvl_parts = [], [], [], []
    for lvl in range(5):
        c = cls_list[lvl]
        r = reg_list[lvl]
        s = jax.nn.sigmoid(jnp.transpose(c, (0, 2, 3, 1)).reshape(B, -1))
        rr = jnp.transpose(r, (0, 2, 3, 1)).reshape(B, -1, 4)
        a = jnp.asarray(_GRIDS[lvl])
        n = s.shape[1]
        if 0 < _PRE_NMS < n:
            s, idx = jax.lax.top_k(s, _PRE_NMS)
            rr = jnp.take_along_axis(rr, idx[..., None], axis=1)
            aa = a[idx]
            n = _PRE_NMS
        else:
            aa = jnp.broadcast_to(a[None], (B, n, 4))
        s_parts.append(s)
        d_parts.append(rr)
        a_parts.append(aa)
        lvl_parts.append(np.full((n,), float(lvl), dtype=np.float32))

    scores = jnp.concatenate(s_parts, axis=1)
    deltas = jnp.concatenate(d_parts, axis=1)
    anchs = jnp.concatenate(a_parts, axis=1)
    levels = np.concatenate(lvl_parts)

    pad = _N_PAD - _N_REAL
    scores_p = jnp.pad(scores, ((0, 0), (0, pad)), constant_values=-3.0)
    deltas_p = jnp.pad(deltas, ((0, 0), (0, pad), (0, 0)))
    anchs_p = jnp.pad(anchs, ((0, 0), (0, pad), (0, 0)))
    levels_p = jnp.asarray(
        np.pad(levels, (0, pad)).reshape(_NB, _BLK))

    def fld(arr, j):
        return arr[..., j].reshape(B, _NB, _BLK)

    sP = scores_p.reshape(B, _NB, _BLK)
    dec = jax.vmap(_decode_call,
                   in_axes=(0, 0, 0, 0, 0, 0, 0, 0, 0, None))(
        fld(anchs_p, 0), fld(anchs_p, 1), fld(anchs_p, 2), fld(anchs_p, 3),
        fld(deltas_p, 0), fld(deltas_p, 1), fld(deltas_p, 2), fld(deltas_p, 3),
        sP, levels_p)
    x1, y1, x2, y2, nx1, ny1, nx2, ny2, sm = [
        t.reshape(B, _N_PAD) for t in dec]

    order = jnp.argsort(-sm, axis=1)

    def g(t):
        return jnp.take_along_axis(t, order, axis=1)

    sm_s = g(sm)
    nx1_s, ny1_s, nx2_s, ny2_s = g(nx1), g(ny1), g(nx2), g(ny2)
    x1_s, y1_s, x2_s, y2_s = g(x1), g(y1), g(x2), g(y2)
    s_s = g(scores_p)

    keep = jax.vmap(_nms_call)(
        nx1_s.reshape(B, _NB, _BLK), ny1_s.reshape(B, _NB, _BLK),
        nx2_s.reshape(B, _NB, _BLK), ny2_s.reshape(B, _NB, _BLK))
    keep = keep.reshape(B, _N_PAD) > 0.5

    valid_s = sm_s > -0.5
    kv = keep & valid_s
    idx7 = jnp.arange(_N_PAD)
    key = jnp.where(kv, idx7[None], _N_PAD + idx7[None])
    perm = jnp.argsort(key, axis=1)[:, :_MAX_PER_IMG]
    ok = jnp.take_along_axis(kv, perm, axis=1)

    def gp(t):
        return jnp.take_along_axis(t, perm, axis=1)

    out_b = jnp.stack([gp(x1_s), gp(y1_s), gp(x2_s), gp(y2_s)], axis=-1)
    out_b = jnp.where(ok[..., None], out_b, 0.0)
    out_s = jnp.where(ok, gp(s_s), 0.0)
    return jnp.concatenate([out_b, out_s[..., None]], axis=-1)
